# baseline (speedup 1.0000x reference)
import functools

import jax
import jax.numpy as jnp
from jax import lax
from jax.experimental import pallas as pl
from jax.experimental.pallas import tpu as pltpu
from jax.experimental.pallas import tpu_sc as plsc

NC = 2
NS = 16
K = 128

NUM_STEPS = 3
NUM_INTER_STEPS = 2


def _transform_body(h_ref, w_ref, b_ref, out_ref):
    out_ref[0] = (
        jnp.dot(h_ref[...], w_ref[0], preferred_element_type=jnp.float32)
        + b_ref[0, 0]
    )


def _make_transform(n, d, nt, bn):
    nb = n // bn
    return pl.pallas_call(
        _transform_body,
        grid=(nb, nt),
        in_specs=[
            pl.BlockSpec((bn, d), lambda j, t: (j, 0)),
            pl.BlockSpec((1, d, d), lambda j, t: (t, 0, 0)),
            pl.BlockSpec((1, 1, d), lambda j, t: (t, 0, 0)),
        ],
        out_specs=pl.BlockSpec((1, bn, d), lambda j, t: (t, j, 0)),
        out_shape=jax.ShapeDtypeStruct((nt, n, d), jnp.float32),
    )


def _gru_body(a2_ref, h_ref, wg_ref, ug_ref, bg_ref, out_ref):
    d = h_ref.shape[1]
    a_in = a2_ref[0]
    a_out = a2_ref[1]
    h = h_ref[...]
    ga = (
        jnp.dot(a_in, wg_ref[:d], preferred_element_type=jnp.float32)
        + jnp.dot(a_out, wg_ref[d:], preferred_element_type=jnp.float32)
        + bg_ref[0]
    )
    gh = jnp.dot(h, ug_ref[...], preferred_element_type=jnp.float32)
    z = jax.nn.sigmoid(ga[:, :d] + gh[:, :d])
    r = jax.nn.sigmoid(ga[:, d:2 * d] + gh[:, d:2 * d])
    ht = jnp.tanh(ga[:, 2 * d:] + r * gh[:, 2 * d:])
    out_ref[...] = (1.0 - z) * h + z * ht


def _make_gru(n, d, bn):
    nb = n // bn
    return pl.pallas_call(
        _gru_body,
        grid=(nb,),
        in_specs=[
            pl.BlockSpec((2, bn, d), lambda j: (0, j, 0)),
            pl.BlockSpec((bn, d), lambda j: (j, 0)),
            pl.BlockSpec((2 * d, 3 * d), lambda j: (0, 0)),
            pl.BlockSpec((d, 3 * d), lambda j: (0, 0)),
            pl.BlockSpec((1, 3 * d), lambda j: (0, 0)),
        ],
        out_specs=pl.BlockSpec((bn, d), lambda j: (j, 0)),
        out_shape=jax.ShapeDtypeStruct((n, d), jnp.float32),
    )


def _imp_body(h_ref, x_ref, w1_ref, b1_ref, w2_ref, b2_ref, out_ref):
    d = h_ref.shape[1]
    t1 = jnp.tanh(
        jnp.dot(h_ref[...], w1_ref[:d], preferred_element_type=jnp.float32)
        + jnp.dot(x_ref[...], w1_ref[d:], preferred_element_type=jnp.float32)
        + b1_ref[0]
    )
    out_ref[...] = jax.nn.sigmoid(
        jnp.dot(t1, w2_ref[...], preferred_element_type=jnp.float32) + b2_ref[0]
    )


def _make_imp(n, d, ann, hid, bn):
    nb = n // bn
    return pl.pallas_call(
        _imp_body,
        grid=(nb,),
        in_specs=[
            pl.BlockSpec((bn, d), lambda j: (j, 0)),
            pl.BlockSpec((bn, ann), lambda j: (j, 0)),
            pl.BlockSpec((d + ann, hid), lambda j: (0, 0)),
            pl.BlockSpec((1, hid), lambda j: (0, 0)),
            pl.BlockSpec((hid, 1), lambda j: (0, 0)),
            pl.BlockSpec((1, 1), lambda j: (0, 0)),
        ],
        out_specs=pl.BlockSpec((bn, 1), lambda j: (j, 0)),
        out_shape=jax.ShapeDtypeStruct((n, 1), jnp.float32),
    )


def _ctx_body(h_ref, x_ref, wc_ref, bc_ref, out_ref):
    d = h_ref.shape[1]
    out_ref[...] = jnp.tanh(
        jnp.dot(h_ref[...], wc_ref[:d], preferred_element_type=jnp.float32)
        + jnp.dot(x_ref[...], wc_ref[d:], preferred_element_type=jnp.float32)
        + bc_ref[0]
    )


def _make_ctx(n, d, ann, cdim, bn):
    nb = n // bn
    return pl.pallas_call(
        _ctx_body,
        grid=(nb,),
        in_specs=[
            pl.BlockSpec((bn, d), lambda j: (j, 0)),
            pl.BlockSpec((bn, ann), lambda j: (j, 0)),
            pl.BlockSpec((d + ann, cdim), lambda j: (0, 0)),
            pl.BlockSpec((1, cdim), lambda j: (0, 0)),
        ],
        out_specs=pl.BlockSpec((bn, cdim), lambda j: (j, 0)),
        out_shape=jax.ShapeDtypeStruct((n, cdim), jnp.float32),
    )


def _make_propagate(n, d, e_pad, acc_rows):
    per_tile = e_pad // NS
    zrows = acc_rows // NS

    mesh = plsc.VectorSubcoreMesh(
        core_axis_name="c", subcore_axis_name="s",
        num_cores=NC, num_subcores=NS,
    )

    @functools.partial(
        pl.kernel,
        out_type=jax.ShapeDtypeStruct((2, acc_rows, d), jnp.float32),
        mesh=mesh,
        scratch_types=[
            pltpu.VMEM((2 * K,), jnp.int32),
            pltpu.VMEM((K, d), jnp.float32),
            pltpu.VMEM((8, d), jnp.float32),
            pltpu.VMEM_SHARED((acc_rows, d), jnp.float32),
            pltpu.SemaphoreType.DMA,
        ],
    )
    def prop(table, cidx, out, idx_v, rows_v, zbuf, acc, sem):
        c = lax.axis_index("c")
        s = lax.axis_index("s")
        zero16 = jnp.zeros((16,), jnp.float32)

        for r in range(8):
            for l in range(d // 16):
                zbuf[r, pl.ds(l * 16, 16)] = zero16

        def zcopy(r, carry):
            pltpu.sync_copy(zbuf, acc.at[pl.ds(s * zrows + r * 8, 8)])
            return carry

        lax.fori_loop(0, zrows // 8, zcopy, 0)
        plsc.subcore_barrier()

        base = (c * e_pad + s * per_tile) * 2

        def chunk(i, carry):
            off = base + i * (2 * K)
            pltpu.sync_copy(cidx.at[pl.ds(off, 2 * K)], idx_v)
            pltpu.async_copy(table.at[idx_v.at[pl.ds(0, K)]], rows_v,
                             sem).wait()
            pltpu.sync_copy(rows_v, acc.at[idx_v.at[pl.ds(K, K)]], add=True)
            return carry

        lax.fori_loop(0, per_tile // K, chunk, 0)
        plsc.subcore_barrier()
        pltpu.sync_copy(
            acc.at[pl.ds(s * zrows, zrows)],
            out.at[c, pl.ds(s * zrows, zrows)],
        )

    return prop


def kernel(x, edge_index, edge_type, W_out, b_out, W_in, b_in,
           Wg, Ug, bg, Wi1, bi1, wi2, bi2, Wc, bc):
    n, ann = x.shape
    t = W_out.shape[0]
    d = W_out.shape[-1]
    e = edge_index.shape[1]
    hid = Wi1.shape[1]
    cdim = Wc.shape[1]

    src = edge_index[0].astype(jnp.int32)
    dst = edge_index[1].astype(jnp.int32)
    et = edge_type.astype(jnp.int32)

    fwd_g = et * n + src
    bwd_g = t * n + et * n + dst

    e_pad = -(-e // (NS * K * 4)) * (NS * K * 4)
    acc_rows = -(-(n + 1) // (8 * NS)) * (8 * NS)
    pad = e_pad - e

    def _pad(a, v):
        return jnp.pad(a, (0, pad), constant_values=v)

    gidx = jnp.concatenate([_pad(fwd_g, 0), _pad(bwd_g, 0)])
    sidx = jnp.concatenate([_pad(dst, n), _pad(src, n)])
    cidx = jnp.stack(
        [gidx.reshape(-1, K), sidx.reshape(-1, K)], axis=1).reshape(-1)

    W_cat = jnp.concatenate([W_out, W_in], axis=0)
    b_cat = jnp.concatenate([b_out, b_in], axis=0).reshape(2 * t, 1, d)
    bg2 = bg.reshape(1, -1)
    bi1_2 = bi1.reshape(1, -1)
    bi2_2 = bi2.reshape(1, -1)
    bc2 = bc.reshape(1, -1)

    bn = 2000
    transform = _make_transform(n, d, 2 * t, bn)
    prop = _make_propagate(n, d, e_pad, acc_rows)
    gru = _make_gru(n, d, bn)
    imp_fn = _make_imp(n, d, ann, hid, bn)
    ctx_fn = _make_ctx(n, d, ann, cdim, bn)

    h = jnp.pad(x, ((0, 0), (0, d - ann)))
    imps = []
    for step in range(NUM_STEPS):
        for _ in range(NUM_INTER_STEPS):
            table = transform(h, W_cat, b_cat).reshape(2 * t * n, d)
            a2 = prop(table, cidx)
            h = gru(a2, h, Wg, Ug, bg2)
        if step != NUM_STEPS - 1:
            imps.append(imp_fn(h, x, Wi1, bi1_2, wi2, bi2_2))
    context = ctx_fn(h, x, Wc, bc2)
    return context, jnp.stack(imps)


# speedup vs baseline: 1.5639x; 1.5639x over previous
import functools

import jax
import jax.numpy as jnp
from jax import lax
from jax.experimental import pallas as pl
from jax.experimental.pallas import tpu as pltpu
from jax.experimental.pallas import tpu_sc as plsc

NC = 2
NS = 16
K = 128

NUM_STEPS = 3
NUM_INTER_STEPS = 2


def _transform_body(h_ref, w_ref, b_ref, out_ref):
    out_ref[0] = (
        jnp.dot(h_ref[...], w_ref[0], preferred_element_type=jnp.float32)
        + b_ref[0, 0]
    )


def _make_transform(n, d, nt, bn):
    nb = n // bn
    return pl.pallas_call(
        _transform_body,
        grid=(nb, nt),
        in_specs=[
            pl.BlockSpec((bn, d), lambda j, t: (j, 0)),
            pl.BlockSpec((1, d, d), lambda j, t: (t, 0, 0)),
            pl.BlockSpec((1, 1, d), lambda j, t: (t, 0, 0)),
        ],
        out_specs=pl.BlockSpec((1, bn, d), lambda j, t: (t, j, 0)),
        out_shape=jax.ShapeDtypeStruct((nt, n, d), jnp.float32),
    )


def _gru_body(a2_ref, h_ref, wg_ref, ug_ref, bg_ref, out_ref):
    d = h_ref.shape[1]
    a_in = a2_ref[0]
    a_out = a2_ref[1]
    h = h_ref[...]
    ga = (
        jnp.dot(a_in, wg_ref[:d], preferred_element_type=jnp.float32)
        + jnp.dot(a_out, wg_ref[d:], preferred_element_type=jnp.float32)
        + bg_ref[0]
    )
    gh = jnp.dot(h, ug_ref[...], preferred_element_type=jnp.float32)
    z = jax.nn.sigmoid(ga[:, :d] + gh[:, :d])
    r = jax.nn.sigmoid(ga[:, d:2 * d] + gh[:, d:2 * d])
    ht = jnp.tanh(ga[:, 2 * d:] + r * gh[:, 2 * d:])
    out_ref[...] = (1.0 - z) * h + z * ht


def _make_gru(n, d, bn):
    nb = n // bn
    return pl.pallas_call(
        _gru_body,
        grid=(nb,),
        in_specs=[
            pl.BlockSpec((2, bn, d), lambda j: (0, j, 0)),
            pl.BlockSpec((bn, d), lambda j: (j, 0)),
            pl.BlockSpec((2 * d, 3 * d), lambda j: (0, 0)),
            pl.BlockSpec((d, 3 * d), lambda j: (0, 0)),
            pl.BlockSpec((1, 3 * d), lambda j: (0, 0)),
        ],
        out_specs=pl.BlockSpec((bn, d), lambda j: (j, 0)),
        out_shape=jax.ShapeDtypeStruct((n, d), jnp.float32),
    )


def _imp_body(h_ref, x_ref, w1_ref, b1_ref, w2_ref, b2_ref, out_ref):
    d = h_ref.shape[1]
    t1 = jnp.tanh(
        jnp.dot(h_ref[...], w1_ref[:d], preferred_element_type=jnp.float32)
        + jnp.dot(x_ref[...], w1_ref[d:], preferred_element_type=jnp.float32)
        + b1_ref[0]
    )
    out_ref[...] = jax.nn.sigmoid(
        jnp.dot(t1, w2_ref[...], preferred_element_type=jnp.float32) + b2_ref[0]
    )


def _make_imp(n, d, ann, hid, bn):
    nb = n // bn
    return pl.pallas_call(
        _imp_body,
        grid=(nb,),
        in_specs=[
            pl.BlockSpec((bn, d), lambda j: (j, 0)),
            pl.BlockSpec((bn, ann), lambda j: (j, 0)),
            pl.BlockSpec((d + ann, hid), lambda j: (0, 0)),
            pl.BlockSpec((1, hid), lambda j: (0, 0)),
            pl.BlockSpec((hid, 1), lambda j: (0, 0)),
            pl.BlockSpec((1, 1), lambda j: (0, 0)),
        ],
        out_specs=pl.BlockSpec((bn, 1), lambda j: (j, 0)),
        out_shape=jax.ShapeDtypeStruct((n, 1), jnp.float32),
    )


def _ctx_body(h_ref, x_ref, wc_ref, bc_ref, out_ref):
    d = h_ref.shape[1]
    out_ref[...] = jnp.tanh(
        jnp.dot(h_ref[...], wc_ref[:d], preferred_element_type=jnp.float32)
        + jnp.dot(x_ref[...], wc_ref[d:], preferred_element_type=jnp.float32)
        + bc_ref[0]
    )


def _make_ctx(n, d, ann, cdim, bn):
    nb = n // bn
    return pl.pallas_call(
        _ctx_body,
        grid=(nb,),
        in_specs=[
            pl.BlockSpec((bn, d), lambda j: (j, 0)),
            pl.BlockSpec((bn, ann), lambda j: (j, 0)),
            pl.BlockSpec((d + ann, cdim), lambda j: (0, 0)),
            pl.BlockSpec((1, cdim), lambda j: (0, 0)),
        ],
        out_specs=pl.BlockSpec((bn, cdim), lambda j: (j, 0)),
        out_shape=jax.ShapeDtypeStruct((n, cdim), jnp.float32),
    )


def _make_propagate(n, d, e_pad, acc_rows):
    per_tile = e_pad // NS
    zrows = acc_rows // NS

    mesh = plsc.VectorSubcoreMesh(
        core_axis_name="c", subcore_axis_name="s",
        num_cores=NC, num_subcores=NS,
    )

    @functools.partial(
        pl.kernel,
        out_type=jax.ShapeDtypeStruct((2, acc_rows, d), jnp.float32),
        mesh=mesh,
        scratch_types=[
            pltpu.VMEM((K,), jnp.int32),
            pltpu.VMEM((1, K), jnp.int32),
            pltpu.VMEM((K, d), jnp.float32),
            pltpu.VMEM((8, d), jnp.float32),
            pltpu.VMEM_SHARED((acc_rows, d), jnp.float32),
            pltpu.SemaphoreType.DMA,
        ],
    )
    def prop(table, gidx, sidx, out, gidx_v, sidx_v, rows_v, zbuf, acc, sem):
        c = lax.axis_index("c")
        s = lax.axis_index("s")
        zero16 = jnp.zeros((16,), jnp.float32)

        for r in range(8):
            for l in range(d // 16):
                zbuf[r, pl.ds(l * 16, 16)] = zero16

        def zcopy(r, carry):
            pltpu.sync_copy(zbuf, acc.at[pl.ds(s * zrows + r * 8, 8)])
            return carry

        lax.fori_loop(0, zrows // 8, zcopy, 0)
        plsc.subcore_barrier()

        base = c * e_pad + s * per_tile

        def chunk(i, carry):
            off = base + i * K
            pltpu.sync_copy(gidx.at[pl.ds(off, K)], gidx_v)
            pltpu.sync_copy(sidx.at[pl.ds(off, K)], sidx_v.at[0])
            pltpu.async_copy(table.at[gidx_v], rows_v, sem).wait()
            pltpu.sync_copy(rows_v, acc.at[sidx_v.at[0]], add=True)
            return carry

        lax.fori_loop(0, per_tile // K, chunk, 0)
        plsc.subcore_barrier()
        pltpu.sync_copy(
            acc.at[pl.ds(s * zrows, zrows)],
            out.at[c, pl.ds(s * zrows, zrows)],
        )

    return prop


def kernel(x, edge_index, edge_type, W_out, b_out, W_in, b_in,
           Wg, Ug, bg, Wi1, bi1, wi2, bi2, Wc, bc):
    n, ann = x.shape
    t = W_out.shape[0]
    d = W_out.shape[-1]
    e = edge_index.shape[1]
    hid = Wi1.shape[1]
    cdim = Wc.shape[1]

    src = edge_index[0].astype(jnp.int32)
    dst = edge_index[1].astype(jnp.int32)
    et = edge_type.astype(jnp.int32)

    fwd_g = et * n + src
    bwd_g = t * n + et * n + dst

    e_pad = -(-e // (NS * K)) * (NS * K)
    acc_rows = -(-(n + 1) // (8 * NS)) * (8 * NS)
    pad = e_pad - e

    def _pad(a, v):
        return jnp.pad(a, (0, pad), constant_values=v)

    gidx = jnp.concatenate([_pad(fwd_g, 0), _pad(bwd_g, 0)])
    sidx = jnp.concatenate([_pad(dst, n), _pad(src, n)])

    W_cat = jnp.concatenate([W_out, W_in], axis=0)
    b_cat = jnp.concatenate([b_out, b_in], axis=0).reshape(2 * t, 1, d)
    bg2 = bg.reshape(1, -1)
    bi1_2 = bi1.reshape(1, -1)
    bi2_2 = bi2.reshape(1, -1)
    bc2 = bc.reshape(1, -1)

    bn = 2000
    transform = _make_transform(n, d, 2 * t, bn)
    prop = _make_propagate(n, d, e_pad, acc_rows)
    gru = _make_gru(n, d, bn)
    imp_fn = _make_imp(n, d, ann, hid, bn)
    ctx_fn = _make_ctx(n, d, ann, cdim, bn)

    h = jnp.pad(x, ((0, 0), (0, d - ann)))
    imps = []
    for step in range(NUM_STEPS):
        for _ in range(NUM_INTER_STEPS):
            table = transform(h, W_cat, b_cat).reshape(2 * t * n, d)
            a2 = prop(table, gidx, sidx)
            h = gru(a2, h, Wg, Ug, bg2)
        if step != NUM_STEPS - 1:
            imps.append(imp_fn(h, x, Wi1, bi1_2, wi2, bi2_2))
    context = ctx_fn(h, x, Wc, bc2)
    return context, jnp.stack(imps)
